# Initial kernel scaffold; baseline (speedup 1.0000x reference)
import jax
import jax.numpy as jnp
from jax.experimental import pallas as pl


def kernel(node_feats, edge_index, edge_feats, W_src, b_src, W_dst, b_dst, W_edge, b_edge, gamma_m, beta_m, gamma_n, beta_n):
    raise NotImplementedError("write your pallas kernel here")


# speedup vs baseline: 1.3748x; 1.3748x over previous
import functools

import jax
import jax.numpy as jnp
from jax import lax
from jax.experimental import pallas as pl
from jax.experimental.pallas import tpu as pltpu
from jax.experimental.pallas import tpu_sc as plsc

N = 10000
E = 320000
D = 128
D2 = 256
EPS = 1e-5

NC = 2
NS = 16
NW = NC * NS
EPW = E // NW
GC = 80
N_GCHUNK = EPW // GC

E_PER_CORE = E // NC
EPT = E_PER_CORE // NS
SC2 = 80
N_SCHUNK = EPT // SC2
ROWS_PER_TILE = N // NS


def _node_proj_body(nf_ref, w_ref, b_ref, hs_ref, hd_ref):
    nf = nf_ref[...]
    w = w_ref[...]
    b = b_ref[...]
    hs_ref[...] = jnp.dot(nf, w[:, :D2], preferred_element_type=jnp.float32) + b[:, :D2]
    hd_ref[...] = jnp.dot(nf, w[:, D2:], preferred_element_type=jnp.float32) + b[:, D2:]


def _edge_proj_body(f_ref, w_ref, b_ref, out_ref):
    out_ref[...] = (
        jnp.dot(f_ref[...], w_ref[...], preferred_element_type=jnp.float32) + b_ref[...]
    )


def _gate_body(m_ref, stats_ref, gm_ref, bm_ref, msg_ref):
    stats = stats_ref[...]
    ssum = jnp.sum(stats[:, 0, :], axis=0)
    ssq = jnp.sum(stats[:, 1, :], axis=0)
    mean = ssum / E
    var = ssq / E - mean * mean
    rstd = lax.rsqrt(var + EPS)
    scale = rstd * gm_ref[0]
    shift = bm_ref[0] - mean * scale
    mhat = m_ref[...] * scale + shift
    h_f = mhat[:, :D]
    h_s = mhat[:, D:]
    msg_ref[...] = jax.nn.sigmoid(h_f) * jax.nn.softplus(h_s)


def _final_body(nf_ref, hp_ref, gn_ref, bn_ref, out_ref):
    h = hp_ref[0] + hp_ref[1]
    mean = jnp.mean(h, axis=0, keepdims=True)
    var = jnp.mean((h - mean) ** 2, axis=0, keepdims=True)
    rstd = lax.rsqrt(var + EPS)
    hn = (h - mean) * rstd * gn_ref[0] + bn_ref[0]
    out_ref[...] = jax.nn.softplus(nf_ref[...] + hn)


def _sc_mesh():
    return plsc.VectorSubcoreMesh(
        core_axis_name="c", subcore_axis_name="s", num_cores=NC, num_subcores=NS
    )


def _gather_pass(hs, hd, src, dst, ep):
    @functools.partial(
        pl.kernel,
        out_type=[
            jax.ShapeDtypeStruct((E, D2), jnp.float32),
            jax.ShapeDtypeStruct((NW, 2, D2), jnp.float32),
        ],
        mesh=_sc_mesh(),
        scratch_types=[
            pltpu.VMEM((GC,), jnp.int32),
            pltpu.VMEM((GC,), jnp.int32),
            pltpu.VMEM((GC, D2), jnp.float32),
            pltpu.VMEM((GC, D2), jnp.float32),
            pltpu.VMEM((GC, D2), jnp.float32),
            pltpu.VMEM((D2,), jnp.float32),
            pltpu.VMEM((D2,), jnp.float32),
            pltpu.SemaphoreType.DMA,
            pltpu.SemaphoreType.DMA,
            pltpu.SemaphoreType.DMA,
        ],
    )
    def k(hs_hbm, hd_hbm, src_hbm, dst_hbm, ep_hbm, m_hbm, stats_hbm,
          ia, ib, ra, rb, rc, accs, accq, s0, s1, s2):
        cid = lax.axis_index("c")
        sid = lax.axis_index("s")
        wid = sid * NC + cid
        base0 = wid * EPW

        zero = jnp.zeros((16,), jnp.float32)
        for g in range(D2 // 16):
            accs[pl.ds(g * 16, 16)] = zero
            accq[pl.ds(g * 16, 16)] = zero

        def chunk_body(ch, carry):
            cb = pl.multiple_of(base0 + ch * GC, 8)
            pltpu.sync_copy(src_hbm.at[pl.ds(cb, GC)], ia)
            pltpu.sync_copy(dst_hbm.at[pl.ds(cb, GC)], ib)
            cpa = pltpu.async_copy(hs_hbm.at[ia], ra, s0)
            cpb = pltpu.async_copy(hd_hbm.at[ib], rb, s1)
            cpc = pltpu.async_copy(ep_hbm.at[pl.ds(cb, GC)], rc, s2)
            cpa.wait()
            cpb.wait()
            cpc.wait()

            def row_body(r, c2):
                for g in range(D2 // 16):
                    sl = pl.ds(g * 16, 16)
                    v = ra[r, sl] + rb[r, sl] + rc[r, sl]
                    ra[r, sl] = v
                    plsc.addupdate(accs.at[sl], v)
                    plsc.addupdate(accq.at[sl], v * v)
                return c2

            lax.fori_loop(0, GC, row_body, 0)
            pltpu.sync_copy(ra, m_hbm.at[pl.ds(cb, GC)])
            return carry

        lax.fori_loop(0, N_GCHUNK, chunk_body, 0)
        pltpu.sync_copy(accs, stats_hbm.at[wid, 0])
        pltpu.sync_copy(accq, stats_hbm.at[wid, 1])

    return k(hs, hd, src, dst, ep)


def _scatter_pass(msg, dst, zero_init):
    @functools.partial(
        pl.kernel,
        out_type=jax.ShapeDtypeStruct((NC, N, D), jnp.float32),
        mesh=_sc_mesh(),
        scratch_types=[
            pltpu.VMEM((SC2,), jnp.int32),
            pltpu.VMEM((SC2, D), jnp.float32),
            pltpu.VMEM_SHARED((N, D), jnp.float32),
        ],
    )
    def k(msg_hbm, dst_hbm, zero_hbm, out_hbm, idx_v, buf, acc_sh):
        cid = lax.axis_index("c")
        sid = lax.axis_index("s")

        @pl.when(sid == 0)
        def _():
            pltpu.sync_copy(zero_hbm, acc_sh)

        plsc.subcore_barrier()

        base0 = cid * E_PER_CORE + sid * EPT

        def chunk_body(ch, carry):
            cb = pl.multiple_of(base0 + ch * SC2, 8)
            pltpu.sync_copy(dst_hbm.at[pl.ds(cb, SC2)], idx_v)
            pltpu.sync_copy(msg_hbm.at[pl.ds(cb, SC2)], buf)
            pltpu.sync_copy(buf, acc_sh.at[idx_v], add=True)
            return carry

        lax.fori_loop(0, N_SCHUNK, chunk_body, 0)
        plsc.subcore_barrier()
        rb = sid * 624

        @pl.when(sid < NS - 1)
        def _():
            pltpu.sync_copy(
                acc_sh.at[pl.ds(rb, 624)], out_hbm.at[cid, pl.ds(rb, 624)]
            )

        @pl.when(sid == NS - 1)
        def _():
            pltpu.sync_copy(
                acc_sh.at[pl.ds(15 * 624, N - 15 * 624)],
                out_hbm.at[cid, pl.ds(15 * 624, N - 15 * 624)],
            )

    return k(msg, dst, zero_init)


def kernel(node_feats, edge_index, edge_feats, W_src, b_src, W_dst, b_dst,
           W_edge, b_edge, gamma_m, beta_m, gamma_n, beta_n):
    src = edge_index[0].astype(jnp.int32)
    dst = edge_index[1].astype(jnp.int32)

    w_cat = jnp.concatenate([W_src.T, W_dst.T], axis=1)
    b_cat = jnp.concatenate([b_src, b_dst]).reshape(1, 2 * D2)

    hs, hd = pl.pallas_call(
        _node_proj_body,
        out_shape=[
            jax.ShapeDtypeStruct((N, D2), jnp.float32),
            jax.ShapeDtypeStruct((N, D2), jnp.float32),
        ],
    )(node_feats, w_cat, b_cat)

    EB = 4000
    ep = pl.pallas_call(
        _edge_proj_body,
        grid=(E // EB,),
        in_specs=[
            pl.BlockSpec((EB, 16), lambda i: (i, 0)),
            pl.BlockSpec((16, D2), lambda i: (0, 0)),
            pl.BlockSpec((1, D2), lambda i: (0, 0)),
        ],
        out_specs=pl.BlockSpec((EB, D2), lambda i: (i, 0)),
        out_shape=jax.ShapeDtypeStruct((E, D2), jnp.float32),
    )(edge_feats, W_edge.T, b_edge.reshape(1, D2))

    m, stats = _gather_pass(hs, hd, src, dst, ep)

    msg = pl.pallas_call(
        _gate_body,
        grid=(E // EB,),
        in_specs=[
            pl.BlockSpec((EB, D2), lambda i: (i, 0)),
            pl.BlockSpec((NW, 2, D2), lambda i: (0, 0, 0)),
            pl.BlockSpec((1, D2), lambda i: (0, 0)),
            pl.BlockSpec((1, D2), lambda i: (0, 0)),
        ],
        out_specs=pl.BlockSpec((EB, D), lambda i: (i, 0)),
        out_shape=jax.ShapeDtypeStruct((E, D), jnp.float32),
    )(m, stats, gamma_m.reshape(1, D2), beta_m.reshape(1, D2))

    zero_init = jnp.zeros((N, D), jnp.float32)
    hpart = _scatter_pass(msg, dst, zero_init)

    out = pl.pallas_call(
        _final_body,
        out_shape=jax.ShapeDtypeStruct((N, D), jnp.float32),
    )(node_feats, hpart, gamma_n.reshape(1, D), beta_n.reshape(1, D))

    return out


# speedup vs baseline: 2.3508x; 1.7099x over previous
import functools

import jax
import jax.numpy as jnp
from jax import lax
from jax.experimental import pallas as pl
from jax.experimental.pallas import tpu as pltpu
from jax.experimental.pallas import tpu_sc as plsc

N = 10000
E = 320000
D = 128
D2 = 256
EPS = 1e-5

NC = 2
NS = 16
NW = NC * NS
EPW = E // NW
GC = 80
N_GCHUNK = EPW // GC

E_PER_CORE = E // NC
EPT = E_PER_CORE // NS
SC2 = 80
N_SCHUNK = EPT // SC2
ROWS_PER_TILE = N // NS


def _node_proj_body(nf_ref, w_ref, b_ref, hs_ref, hd_ref):
    nf = nf_ref[...]
    w = w_ref[...]
    b = b_ref[...]
    hs_ref[...] = jnp.dot(nf, w[:, :D2], preferred_element_type=jnp.float32) + b[:, :D2]
    hd_ref[...] = jnp.dot(nf, w[:, D2:], preferred_element_type=jnp.float32) + b[:, D2:]


def _edge_proj_body(f_ref, w_ref, b_ref, out_ref):
    out_ref[...] = (
        jnp.dot(f_ref[...], w_ref[...], preferred_element_type=jnp.float32) + b_ref[...]
    )


def _gate_body(m_ref, stats_ref, gm_ref, bm_ref, msg_ref):
    stats = stats_ref[...]
    ssum = jnp.sum(stats[:, 0, :], axis=0)
    ssq = jnp.sum(stats[:, 1, :], axis=0)
    mean = ssum / E
    var = ssq / E - mean * mean
    rstd = lax.rsqrt(var + EPS)
    scale = rstd * gm_ref[0]
    shift = bm_ref[0] - mean * scale
    mhat = m_ref[...] * scale + shift
    h_f = mhat[:, :D]
    h_s = mhat[:, D:]
    msg_ref[...] = jax.nn.sigmoid(h_f) * jax.nn.softplus(h_s)


def _final_body(nf_ref, hp_ref, gn_ref, bn_ref, out_ref):
    h = hp_ref[0] + hp_ref[1]
    mean = jnp.mean(h, axis=0, keepdims=True)
    var = jnp.mean((h - mean) ** 2, axis=0, keepdims=True)
    rstd = lax.rsqrt(var + EPS)
    hn = (h - mean) * rstd * gn_ref[0] + bn_ref[0]
    out_ref[...] = jax.nn.softplus(nf_ref[...] + hn)


def _sc_mesh():
    return plsc.VectorSubcoreMesh(
        core_axis_name="c", subcore_axis_name="s", num_cores=NC, num_subcores=NS
    )


def _gather_pass(hs, hd, src, dst, ep):
    NG = D2 // 16

    @functools.partial(
        pl.kernel,
        out_type=[
            jax.ShapeDtypeStruct((E, D2), jnp.float32),
            jax.ShapeDtypeStruct((NW, 2, D2), jnp.float32),
        ],
        mesh=_sc_mesh(),
        scratch_types=[
            pltpu.VMEM((2, GC), jnp.int32),
            pltpu.VMEM((2, GC), jnp.int32),
            pltpu.VMEM((GC, D2), jnp.float32),
            pltpu.VMEM((GC, D2), jnp.float32),
            pltpu.VMEM((GC, D2), jnp.float32),
            pltpu.VMEM((GC, D2), jnp.float32),
            pltpu.VMEM((GC, D2), jnp.float32),
            pltpu.VMEM((GC, D2), jnp.float32),
            pltpu.VMEM((D2,), jnp.float32),
            pltpu.VMEM((D2,), jnp.float32),
        ]
        + [pltpu.SemaphoreType.DMA] * 8,
    )
    def k(hs_hbm, hd_hbm, src_hbm, dst_hbm, ep_hbm, m_hbm, stats_hbm,
          ia, ib, ra0, rb0, rc0, ra1, rb1, rc1, accs, accq,
          sa0, sb0, sc0, sw0, sa1, sb1, sc1, sw1):
        cid = lax.axis_index("c")
        sid = lax.axis_index("s")
        wid = sid * NC + cid
        base0 = wid * EPW

        bufs = ((ra0, rb0, rc0, sa0, sb0, sc0, sw0),
                (ra1, rb1, rc1, sa1, sb1, sc1, sw1))

        zero = jnp.zeros((16,), jnp.float32)
        for g in range(NG):
            accs[pl.ds(g * 16, 16)] = zero
            accq[pl.ds(g * 16, 16)] = zero

        def issue(ch, b, drain):
            ra, rb, rc, sa, sb, sc_, sw = bufs[b]
            if drain is not None:
                @pl.when(drain)
                def _():
                    pltpu.make_async_copy(
                        ra, m_hbm.at[pl.ds(pl.multiple_of(base0, 8), GC)], sw
                    ).wait()
            cb = pl.multiple_of(base0 + ch * GC, 8)
            pltpu.sync_copy(src_hbm.at[pl.ds(cb, GC)], ia.at[b])
            pltpu.sync_copy(dst_hbm.at[pl.ds(cb, GC)], ib.at[b])
            pltpu.async_copy(hs_hbm.at[ia.at[b]], ra, sa)
            pltpu.async_copy(hd_hbm.at[ib.at[b]], rb, sb)
            pltpu.async_copy(ep_hbm.at[pl.ds(cb, GC)], rc, sc_)

        def compute(ch, b):
            ra, rb, rc, sa, sb, sc_, sw = bufs[b]
            lin = ep_hbm.at[pl.ds(pl.multiple_of(base0, 8), GC)]
            pltpu.make_async_copy(lin, ra, sa).wait()
            pltpu.make_async_copy(lin, rb, sb).wait()
            pltpu.make_async_copy(lin, rc, sc_).wait()

            def row_body(r, carry):
                cs = list(carry)
                for g in range(NG):
                    sl = pl.ds(g * 16, 16)
                    v = ra[r, sl] + rb[r, sl] + rc[r, sl]
                    ra[r, sl] = v
                    cs[g] = cs[g] + v
                    cs[NG + g] = cs[NG + g] + v * v
                return tuple(cs)

            init = (jnp.zeros((16,), jnp.float32),) * (2 * NG)
            sums = lax.fori_loop(0, GC, row_body, init)
            for g in range(NG):
                sl = pl.ds(g * 16, 16)
                plsc.addupdate(accs.at[sl], sums[g])
                plsc.addupdate(accq.at[sl], sums[NG + g])
            cb = pl.multiple_of(base0 + ch * GC, 8)
            pltpu.async_copy(ra, m_hbm.at[pl.ds(cb, GC)], sw)

        issue(0, 0, drain=None)

        def body(i, carry):
            @pl.when(i % 2 == 0)
            def _():
                issue(i + 1, 1, drain=i >= 1)
                compute(i, 0)

            @pl.when(i % 2 == 1)
            def _():
                issue(i + 1, 0, drain=i >= 1)
                compute(i, 1)

            return carry

        lax.fori_loop(0, N_GCHUNK - 1, body, 0)
        compute(N_GCHUNK - 1, (N_GCHUNK - 1) % 2)

        for b in (0, 1):
            ra = bufs[b][0]
            sw = bufs[b][6]
            pltpu.make_async_copy(
                ra, m_hbm.at[pl.ds(pl.multiple_of(base0, 8), GC)], sw
            ).wait()
        pltpu.sync_copy(accs, stats_hbm.at[wid, 0])
        pltpu.sync_copy(accq, stats_hbm.at[wid, 1])

    return k(hs, hd, src, dst, ep)


def _scatter_pass(msg, dst, zero_init):
    @functools.partial(
        pl.kernel,
        out_type=jax.ShapeDtypeStruct((NC, N, D), jnp.float32),
        mesh=_sc_mesh(),
        scratch_types=[
            pltpu.VMEM((SC2,), jnp.int32),
            pltpu.VMEM((SC2, D), jnp.float32),
            pltpu.VMEM_SHARED((N, D), jnp.float32),
        ],
    )
    def k(msg_hbm, dst_hbm, zero_hbm, out_hbm, idx_v, buf, acc_sh):
        cid = lax.axis_index("c")
        sid = lax.axis_index("s")

        @pl.when(sid == 0)
        def _():
            pltpu.sync_copy(zero_hbm, acc_sh)

        plsc.subcore_barrier()

        base0 = cid * E_PER_CORE + sid * EPT

        def chunk_body(ch, carry):
            cb = pl.multiple_of(base0 + ch * SC2, 8)
            pltpu.sync_copy(dst_hbm.at[pl.ds(cb, SC2)], idx_v)
            pltpu.sync_copy(msg_hbm.at[pl.ds(cb, SC2)], buf)
            pltpu.sync_copy(buf, acc_sh.at[idx_v], add=True)
            return carry

        lax.fori_loop(0, N_SCHUNK, chunk_body, 0)
        plsc.subcore_barrier()
        rb = sid * 624

        @pl.when(sid < NS - 1)
        def _():
            pltpu.sync_copy(
                acc_sh.at[pl.ds(rb, 624)], out_hbm.at[cid, pl.ds(rb, 624)]
            )

        @pl.when(sid == NS - 1)
        def _():
            pltpu.sync_copy(
                acc_sh.at[pl.ds(15 * 624, N - 15 * 624)],
                out_hbm.at[cid, pl.ds(15 * 624, N - 15 * 624)],
            )

    return k(msg, dst, zero_init)


def kernel(node_feats, edge_index, edge_feats, W_src, b_src, W_dst, b_dst,
           W_edge, b_edge, gamma_m, beta_m, gamma_n, beta_n):
    src = edge_index[0].astype(jnp.int32)
    dst = edge_index[1].astype(jnp.int32)

    w_cat = jnp.concatenate([W_src.T, W_dst.T], axis=1)
    b_cat = jnp.concatenate([b_src, b_dst]).reshape(1, 2 * D2)

    hs, hd = pl.pallas_call(
        _node_proj_body,
        out_shape=[
            jax.ShapeDtypeStruct((N, D2), jnp.float32),
            jax.ShapeDtypeStruct((N, D2), jnp.float32),
        ],
    )(node_feats, w_cat, b_cat)

    EB = 4000
    ep = pl.pallas_call(
        _edge_proj_body,
        grid=(E // EB,),
        in_specs=[
            pl.BlockSpec((EB, 16), lambda i: (i, 0)),
            pl.BlockSpec((16, D2), lambda i: (0, 0)),
            pl.BlockSpec((1, D2), lambda i: (0, 0)),
        ],
        out_specs=pl.BlockSpec((EB, D2), lambda i: (i, 0)),
        out_shape=jax.ShapeDtypeStruct((E, D2), jnp.float32),
    )(edge_feats, W_edge.T, b_edge.reshape(1, D2))

    m, stats = _gather_pass(hs, hd, src, dst, ep)

    msg = pl.pallas_call(
        _gate_body,
        grid=(E // EB,),
        in_specs=[
            pl.BlockSpec((EB, D2), lambda i: (i, 0)),
            pl.BlockSpec((NW, 2, D2), lambda i: (0, 0, 0)),
            pl.BlockSpec((1, D2), lambda i: (0, 0)),
            pl.BlockSpec((1, D2), lambda i: (0, 0)),
        ],
        out_specs=pl.BlockSpec((EB, D), lambda i: (i, 0)),
        out_shape=jax.ShapeDtypeStruct((E, D), jnp.float32),
    )(m, stats, gamma_m.reshape(1, D2), beta_m.reshape(1, D2))

    zero_init = jnp.zeros((N, D), jnp.float32)
    hpart = _scatter_pass(msg, dst, zero_init)

    out = pl.pallas_call(
        _final_body,
        out_shape=jax.ShapeDtypeStruct((N, D), jnp.float32),
    )(node_feats, hpart, gamma_n.reshape(1, D), beta_n.reshape(1, D))

    return out
